# baseline (speedup 1.0000x reference)
import functools

import jax
import jax.numpy as jnp
from jax import lax
from jax.experimental import pallas as pl
from jax.experimental.pallas import tpu as pltpu
from jax.experimental.pallas import tpu_sc as plsc

B = 16384
D = 32
EPS = 1e-05
NC = 2
NS = 16
NW = NC * NS
BPW = B // NW
CHUNK = 128
NCHUNK = BPW // CHUNK
LANES = 16
GROUPS = BPW // LANES

_mesh = plsc.VectorSubcoreMesh(core_axis_name="c", subcore_axis_name="s")


@functools.partial(
    pl.kernel,
    mesh=_mesh,
    compiler_params=pltpu.CompilerParams(
        use_tc_tiling_on_sc=False, needs_layout_passes=False),
    out_type=jax.ShapeDtypeStruct((B,), jnp.float32),
    scratch_types=[
        pltpu.VMEM((NCHUNK, CHUNK), jnp.int32),
        pltpu.VMEM((NCHUNK, CHUNK), jnp.int32),
        pltpu.VMEM((BPW, D), jnp.float32),
        pltpu.VMEM((BPW, D), jnp.float32),
        pltpu.VMEM((BPW,), jnp.float32),
        pltpu.SemaphoreType.DMA,
    ],
)
def _gamma_sc(lidx_hbm, ridx_hbm, table_hbm, out_hbm,
              lidx_v, ridx_v, u_v, v_v, g_v, sem):
    wid = lax.axis_index("s") * NC + lax.axis_index("c")
    row0 = wid * NCHUNK
    pltpu.sync_copy(lidx_hbm.at[pl.ds(row0, NCHUNK)], lidx_v)
    pltpu.sync_copy(ridx_hbm.at[pl.ds(row0, NCHUNK)], ridx_v)
    copies = []
    for j in range(NCHUNK):
        copies.append(pltpu.async_copy(
            table_hbm.at[lidx_v.at[j]], u_v.at[pl.ds(j * CHUNK, CHUNK)], sem))
        copies.append(pltpu.async_copy(
            table_hbm.at[ridx_v.at[j]], v_v.at[pl.ds(j * CHUNK, CHUNK)], sem))
    for c in copies:
        c.wait()

    lanes = lax.iota(jnp.int32, LANES)

    def body(g, carry):
        rows = g * LANES + lanes
        uu = jnp.zeros((LANES,), jnp.float32)
        vv = jnp.zeros((LANES,), jnp.float32)
        uv = jnp.zeros((LANES,), jnp.float32)
        for dcol in range(D):
            dvec = jnp.full((LANES,), dcol, jnp.int32)
            gu = plsc.load_gather(u_v, [rows, dvec])
            gv = plsc.load_gather(v_v, [rows, dvec])
            uu = uu + gu * gu
            vv = vv + gv * gv
            uv = uv + gu * gv
        alpha = 1.0 - uu
        alpha = jnp.where(alpha <= 0.0, EPS, alpha)
        beta = 1.0 - vv
        beta = jnp.where(beta <= 0.0, EPS, beta)
        gamma = 1.0 + 2.0 * (uu - 2.0 * uv + vv) / alpha / beta
        gamma = jnp.maximum(gamma, 1.0)
        g_v[pl.ds(g * LANES, LANES)] = gamma
        return carry

    lax.fori_loop(0, GROUPS, body, 0)
    pltpu.sync_copy(g_v, out_hbm.at[pl.ds(wid * BPW, BPW)])


def _arcosh_body(g_ref, o_ref):
    g = g_ref[...]
    o_ref[...] = jnp.log(g + jnp.sqrt(g * g - 1.0))


def _arcosh(gamma2d):
    return pl.pallas_call(
        _arcosh_body,
        out_shape=jax.ShapeDtypeStruct(gamma2d.shape, jnp.float32),
    )(gamma2d)


def kernel(left_idx, right_idx, table):
    lidx = left_idx.astype(jnp.int32).reshape(B // CHUNK, CHUNK)
    ridx = right_idx.astype(jnp.int32).reshape(B // CHUNK, CHUNK)
    gamma = _gamma_sc(lidx, ridx, table)
    dists = _arcosh(gamma.reshape(128, 128))
    return dists.reshape(B)
